# baseline (speedup 1.0000x reference)
import functools

import jax
import jax.numpy as jnp
from jax import lax
from jax.experimental import pallas as pl
from jax.experimental.pallas import tpu as pltpu
from jax.experimental.pallas import tpu_sc as plsc

_EPS = 1e-8

_NC, _NS, _L = 2, 16, 16
_NW = _NC * _NS
_N = 50000
_V = 100000
_D = 128
_SENT = _V
_NB = 100352
_SLICE = _NB // _NS
_IDXW = 112
_ROWS_W = 14
_PAD_B = _NW * _ROWS_W * _IDXW
_BLK = 1000
_ZCH = 1568


@functools.partial(
    pl.kernel,
    mesh=plsc.VectorSubcoreMesh(core_axis_name="c", subcore_axis_name="s"),
    out_type=jax.ShapeDtypeStruct((_NC * 2 * _NB,), jnp.float32),
    scratch_types=[
        pltpu.VMEM((_ROWS_W, 1, _IDXW), jnp.int32),
        pltpu.VMEM((_IDXW,), jnp.float32),
        pltpu.VMEM((_ZCH,), jnp.float32),
        pltpu.VMEM_SHARED((_NB,), jnp.float32),
        pltpu.VMEM_SHARED((_NB,), jnp.float32),
        pltpu.SemaphoreType.DMA,
    ],
)
def _hist(pos_hbm, neg_hbm, out_hbm, idx3, ones_v, zbuf, hist_p, hist_n,
          sem):
    cid = lax.axis_index("c")
    sid = lax.axis_index("s")
    wid = sid * _NC + cid

    one = jnp.ones((_L,), jnp.float32)
    zero = jnp.zeros((_L,), jnp.float32)
    for t in range(_IDXW // _L):
        ones_v[pl.ds(t * _L, _L)] = one
    for t in range(_ZCH // _L):
        zbuf[pl.ds(t * _L, _L)] = zero

    for hist in (hist_p, hist_n):
        for q in range(_SLICE // _ZCH):
            pltpu.sync_copy(
                zbuf, hist.at[pl.ds(sid * _SLICE + q * _ZCH, _ZCH)])
    plsc.subcore_barrier()

    for hist, src_hbm in ((hist_p, pos_hbm), (hist_n, neg_hbm)):
        pltpu.sync_copy(src_hbm.at[pl.ds(wid * _ROWS_W, _ROWS_W)], idx3)
        for j in range(_ROWS_W):
            pltpu.sync_copy(ones_v, hist.at[idx3.at[j, 0]], add=True)
    plsc.subcore_barrier()

    for li, hist in enumerate((hist_p, hist_n)):
        off = (cid * 2 + li) * _NB + sid * _SLICE
        pltpu.sync_copy(hist.at[pl.ds(sid * _SLICE, _SLICE)],
                        out_hbm.at[pl.ds(off, _SLICE)])


def _dense_body(rows_ref, cnt_hbm, out_ref, cnt_v, sem):
    i = pl.program_id(0)
    pltpu.make_async_copy(
        cnt_hbm.at[pl.ds(i * _BLK, _BLK)], cnt_v, sem).start()
    r = rows_ref[...]
    sq = jnp.sum(r * r, axis=1)
    inv = 1.0 / jnp.maximum(jnp.sqrt(sq), _EPS)
    pltpu.make_async_copy(
        cnt_hbm.at[pl.ds(i * _BLK, _BLK)], cnt_v, sem).wait()
    cp = cnt_v[:, 0]
    cn = cnt_v[:, 1]
    w8 = jnp.concatenate(
        [cp[None, :], (cp * inv)[None, :], cn[None, :], (cn * inv)[None, :],
         jnp.zeros((4, _BLK), jnp.float32)], axis=0)
    part = jnp.dot(w8, r, preferred_element_type=jnp.float32)

    @pl.when(i == 0)
    def _():
        out_ref[...] = part

    @pl.when(i > 0)
    def _():
        out_ref[...] += part


_dense = pl.pallas_call(
    _dense_body,
    grid=(_V // _BLK,),
    in_specs=[
        pl.BlockSpec((_BLK, _D), lambda i: (i, 0)),
        pl.BlockSpec(memory_space=pl.ANY),
    ],
    out_specs=pl.BlockSpec((8, _D), lambda i: (0, 0)),
    out_shape=jax.ShapeDtypeStruct((8, _D), jnp.float32),
    scratch_shapes=[
        pltpu.VMEM((_BLK, 2), jnp.float32),
        pltpu.SemaphoreType.DMA,
    ],
)


def _side_loss(s_vec, w_vec):
    c = s_vec / _N
    cnorm = jnp.maximum(jnp.sqrt(jnp.sum(c * c)), _EPS)
    mean_cos = jnp.dot(w_vec, c) / (_N * cnorm)
    return 2.0 - 2.0 * mean_cos


def kernel(embeddings, positive_nodes, negative_nodes):
    pad = jnp.full((_PAD_B - _N,), _SENT, jnp.int32)
    pos = jnp.concatenate([positive_nodes.astype(jnp.int32), pad])
    neg = jnp.concatenate([negative_nodes.astype(jnp.int32), pad])
    counts = _hist(pos.reshape(-1, 1, _IDXW), neg.reshape(-1, 1, _IDXW))
    c2 = counts.reshape(_NC, 2, _NB).sum(axis=0)[:, :_V].T
    parts = _dense(embeddings, c2)
    pos_loss = _side_loss(parts[0], parts[1])
    neg_loss = _side_loss(parts[2], parts[3])
    return (pos_loss + neg_loss) / 2.0


# speedup vs baseline: 1.1766x; 1.1766x over previous
import functools

import jax
import jax.numpy as jnp
from jax import lax
from jax.experimental import pallas as pl
from jax.experimental.pallas import tpu as pltpu
from jax.experimental.pallas import tpu_sc as plsc

_EPS = 1e-8

_NC, _NS, _L = 2, 16, 16
_NW = _NC * _NS
_N = 50000
_V = 100000
_D = 128
_SENT = _V
_NB = 100352
_SLICE = _NB // _NS
_IDXW = 112
_ROWS_W = 14
_PAD_B = _NW * _ROWS_W * _IDXW
_BLK = 1000
_ZCH = 1568


@functools.partial(
    pl.kernel,
    mesh=plsc.VectorSubcoreMesh(core_axis_name="c", subcore_axis_name="s"),
    out_type=jax.ShapeDtypeStruct((_NC * 2 * _NB,), jnp.float32),
    scratch_types=[
        pltpu.VMEM((_ROWS_W, 1, _IDXW), jnp.int32),
        pltpu.VMEM((_IDXW,), jnp.float32),
        pltpu.VMEM((_ZCH,), jnp.float32),
        pltpu.VMEM_SHARED((_NB,), jnp.float32),
        pltpu.VMEM_SHARED((_NB,), jnp.float32),
        pltpu.SemaphoreType.DMA,
    ],
)
def _hist(pos_hbm, neg_hbm, out_hbm, idx3, ones_v, zbuf, hist_p, hist_n,
          sem):
    cid = lax.axis_index("c")
    sid = lax.axis_index("s")
    wid = sid * _NC + cid

    one = jnp.ones((_L,), jnp.float32)
    zero = jnp.zeros((_L,), jnp.float32)
    for t in range(_IDXW // _L):
        ones_v[pl.ds(t * _L, _L)] = one
    for t in range(_ZCH // _L):
        zbuf[pl.ds(t * _L, _L)] = zero

    for hist in (hist_p, hist_n):
        for q in range(_SLICE // _ZCH):
            pltpu.sync_copy(
                zbuf, hist.at[pl.ds(sid * _SLICE + q * _ZCH, _ZCH)])
    plsc.subcore_barrier()

    for hist, src_hbm in ((hist_p, pos_hbm), (hist_n, neg_hbm)):
        pltpu.sync_copy(src_hbm.at[pl.ds(wid * _ROWS_W, _ROWS_W)], idx3)
        for j in range(_ROWS_W):
            pltpu.sync_copy(ones_v, hist.at[idx3.at[j, 0]], add=True)
    plsc.subcore_barrier()

    for li, hist in enumerate((hist_p, hist_n)):
        off = (cid * 2 + li) * _NB + sid * _SLICE
        pltpu.sync_copy(hist.at[pl.ds(sid * _SLICE, _SLICE)],
                        out_hbm.at[pl.ds(off, _SLICE)])


_DN_S = (((0,), (0,)), ((), ()))


def _dense_body(rows_ref, cnt_hbm, s_ref, w_ref, cnt_v, sem):
    i = pl.program_id(0)
    pltpu.make_async_copy(
        cnt_hbm.at[pl.ds(i * _BLK, _BLK)], cnt_v, sem).start()
    r = rows_ref[...]
    sq = jnp.sum(r * r, axis=1, keepdims=True)
    inv2 = 1.0 / jnp.maximum(jnp.sqrt(sq), _EPS)
    pltpu.make_async_copy(
        cnt_hbm.at[pl.ds(i * _BLK, _BLK)], cnt_v, sem).wait()
    cnt = cnt_v[...]
    s_part = lax.dot_general(cnt, r, _DN_S,
                             preferred_element_type=jnp.float32)
    w_part = lax.dot_general(cnt * inv2, r, _DN_S,
                             preferred_element_type=jnp.float32)

    @pl.when(i == 0)
    def _():
        s_ref[...] = s_part
        w_ref[...] = w_part

    @pl.when(i > 0)
    def _():
        s_ref[...] += s_part
        w_ref[...] += w_part


_dense = pl.pallas_call(
    _dense_body,
    grid=(_V // _BLK,),
    in_specs=[
        pl.BlockSpec((_BLK, _D), lambda i: (i, 0)),
        pl.BlockSpec(memory_space=pl.ANY),
    ],
    out_specs=[
        pl.BlockSpec((2, _D), lambda i: (0, 0)),
        pl.BlockSpec((2, _D), lambda i: (0, 0)),
    ],
    out_shape=[
        jax.ShapeDtypeStruct((2, _D), jnp.float32),
        jax.ShapeDtypeStruct((2, _D), jnp.float32),
    ],
    scratch_shapes=[
        pltpu.VMEM((_BLK, 2), jnp.float32),
        pltpu.SemaphoreType.DMA,
    ],
)


def _side_loss(s_vec, w_vec):
    c = s_vec / _N
    cnorm = jnp.maximum(jnp.sqrt(jnp.sum(c * c)), _EPS)
    mean_cos = jnp.dot(w_vec, c) / (_N * cnorm)
    return 2.0 - 2.0 * mean_cos


def kernel(embeddings, positive_nodes, negative_nodes):
    pad = jnp.full((_PAD_B - _N,), _SENT, jnp.int32)
    pos = jnp.concatenate([positive_nodes.astype(jnp.int32), pad])
    neg = jnp.concatenate([negative_nodes.astype(jnp.int32), pad])
    counts = _hist(pos.reshape(-1, 1, _IDXW), neg.reshape(-1, 1, _IDXW))
    c2 = counts.reshape(_NC, 2, _NB).sum(axis=0)[:, :_V].T
    s_out, w_out = _dense(embeddings, c2)
    pos_loss = _side_loss(s_out[0], w_out[0])
    neg_loss = _side_loss(s_out[1], w_out[1])
    return (pos_loss + neg_loss) / 2.0


# speedup vs baseline: 1.7203x; 1.4621x over previous
import functools

import jax
import jax.numpy as jnp
from jax import lax
from jax.experimental import pallas as pl
from jax.experimental.pallas import tpu as pltpu
from jax.experimental.pallas import tpu_sc as plsc

_EPS = 1e-8

_NC, _NS, _L = 2, 16, 16
_NW = _NC * _NS
_N = 50000
_V = 100000
_D = 128
_SENT = _V
_NB = 100352
_SLICE = _NB // _NS
_IDXW = 112
_ROWS_W = 14
_PAD_B = _NW * _ROWS_W * _IDXW
_BLK = 1000
_ZCH = 1568


@functools.partial(
    pl.kernel,
    mesh=plsc.VectorSubcoreMesh(core_axis_name="c", subcore_axis_name="s"),
    out_type=jax.ShapeDtypeStruct((_NC * 2 * _NB,), jnp.float32),
    scratch_types=[
        pltpu.VMEM((_ROWS_W, 1, _IDXW), jnp.int32),
        pltpu.VMEM((_IDXW,), jnp.float32),
        pltpu.VMEM((_ZCH,), jnp.float32),
        pltpu.VMEM_SHARED((_NB,), jnp.float32),
        pltpu.VMEM_SHARED((_NB,), jnp.float32),
        pltpu.SemaphoreType.DMA,
    ],
)
def _hist(pos_hbm, neg_hbm, out_hbm, idx3, ones_v, zbuf, hist_p, hist_n,
          sem):
    cid = lax.axis_index("c")
    sid = lax.axis_index("s")
    wid = sid * _NC + cid

    one = jnp.ones((_L,), jnp.float32)
    zero = jnp.zeros((_L,), jnp.float32)
    for t in range(_IDXW // _L):
        ones_v[pl.ds(t * _L, _L)] = one
    for t in range(_ZCH // _L):
        zbuf[pl.ds(t * _L, _L)] = zero

    for hist in (hist_p, hist_n):
        for q in range(_SLICE // _ZCH):
            pltpu.sync_copy(
                zbuf, hist.at[pl.ds(sid * _SLICE + q * _ZCH, _ZCH)])
    plsc.subcore_barrier()

    for hist, src_hbm in ((hist_p, pos_hbm), (hist_n, neg_hbm)):
        pltpu.sync_copy(src_hbm.at[pl.ds(wid * _ROWS_W, _ROWS_W)], idx3)
        for j in range(_ROWS_W):
            pltpu.sync_copy(ones_v, hist.at[idx3.at[j, 0]], add=True)
    plsc.subcore_barrier()

    for li, hist in enumerate((hist_p, hist_n)):
        off = (cid * 2 + li) * _NB + sid * _SLICE
        pltpu.sync_copy(hist.at[pl.ds(sid * _SLICE, _SLICE)],
                        out_hbm.at[pl.ds(off, _SLICE)])


_DN_S = (((0,), (0,)), ((), ()))


def _dense_body(rows_ref, cnt_ref, s_ref, w_ref):
    i = pl.program_id(0)
    r = rows_ref[...]
    sq = jnp.sum(r * r, axis=1, keepdims=True)
    inv2 = 1.0 / jnp.maximum(jnp.sqrt(sq), _EPS)
    cnt = cnt_ref[0]
    s_part = lax.dot_general(cnt, r, _DN_S,
                             preferred_element_type=jnp.float32)
    w_part = lax.dot_general(cnt * inv2, r, _DN_S,
                             preferred_element_type=jnp.float32)

    @pl.when(i == 0)
    def _():
        s_ref[...] = s_part
        w_ref[...] = w_part

    @pl.when(i > 0)
    def _():
        s_ref[...] += s_part
        w_ref[...] += w_part


_dense = pl.pallas_call(
    _dense_body,
    grid=(_V // _BLK,),
    in_specs=[
        pl.BlockSpec((_BLK, _D), lambda i: (i, 0)),
        pl.BlockSpec((1, _BLK, 2), lambda i: (i, 0, 0)),
    ],
    out_specs=[
        pl.BlockSpec((2, _D), lambda i: (0, 0)),
        pl.BlockSpec((2, _D), lambda i: (0, 0)),
    ],
    out_shape=[
        jax.ShapeDtypeStruct((2, _D), jnp.float32),
        jax.ShapeDtypeStruct((2, _D), jnp.float32),
    ],
)


def _side_loss(s_vec, w_vec):
    c = s_vec / _N
    cnorm = jnp.maximum(jnp.sqrt(jnp.sum(c * c)), _EPS)
    mean_cos = jnp.dot(w_vec, c) / (_N * cnorm)
    return 2.0 - 2.0 * mean_cos


def kernel(embeddings, positive_nodes, negative_nodes):
    pad = jnp.full((_PAD_B - _N,), _SENT, jnp.int32)
    pos = jnp.concatenate([positive_nodes.astype(jnp.int32), pad])
    neg = jnp.concatenate([negative_nodes.astype(jnp.int32), pad])
    counts = _hist(pos.reshape(-1, 1, _IDXW), neg.reshape(-1, 1, _IDXW))
    c2 = counts.reshape(_NC, 2, _NB).sum(axis=0)[:, :_V].T
    s_out, w_out = _dense(embeddings, c2.reshape(_V // _BLK, _BLK, 2))
    pos_loss = _side_loss(s_out[0], w_out[0])
    neg_loss = _side_loss(s_out[1], w_out[1])
    return (pos_loss + neg_loss) / 2.0


# speedup vs baseline: 2.5355x; 1.4739x over previous
import functools

import jax
import jax.numpy as jnp
from jax import lax
from jax.experimental import pallas as pl
from jax.experimental.pallas import tpu as pltpu
from jax.experimental.pallas import tpu_sc as plsc

_EPS = 1e-8

_NC, _NS, _L = 2, 16, 16
_NW = _NC * _NS
_N = 50000
_V = 100000
_D = 128
_SENT = _V
_NB = 100352
_SLICE = _NB // _NS
_IDXW = 112
_ROWS_W = 14
_PAD_B = _NW * _ROWS_W * _IDXW
_BLK = 10000
_ZCH = 1568


@functools.partial(
    pl.kernel,
    mesh=plsc.VectorSubcoreMesh(core_axis_name="c", subcore_axis_name="s"),
    out_type=jax.ShapeDtypeStruct((_NC * 2 * _NB,), jnp.float32),
    scratch_types=[
        pltpu.VMEM((_ROWS_W, 1, _IDXW), jnp.int32),
        pltpu.VMEM((_IDXW,), jnp.float32),
        pltpu.VMEM((_ZCH,), jnp.float32),
        pltpu.VMEM_SHARED((_NB,), jnp.float32),
        pltpu.VMEM_SHARED((_NB,), jnp.float32),
        pltpu.SemaphoreType.DMA,
    ],
)
def _hist(pos_hbm, neg_hbm, out_hbm, idx3, ones_v, zbuf, hist_p, hist_n,
          sem):
    cid = lax.axis_index("c")
    sid = lax.axis_index("s")
    wid = sid * _NC + cid

    one = jnp.ones((_L,), jnp.float32)
    zero = jnp.zeros((_L,), jnp.float32)
    for t in range(_IDXW // _L):
        ones_v[pl.ds(t * _L, _L)] = one
    for t in range(_ZCH // _L):
        zbuf[pl.ds(t * _L, _L)] = zero

    for hist in (hist_p, hist_n):
        for q in range(_SLICE // _ZCH):
            pltpu.sync_copy(
                zbuf, hist.at[pl.ds(sid * _SLICE + q * _ZCH, _ZCH)])
    plsc.subcore_barrier()

    for hist, src_hbm in ((hist_p, pos_hbm), (hist_n, neg_hbm)):
        pltpu.sync_copy(src_hbm.at[pl.ds(wid * _ROWS_W, _ROWS_W)], idx3)
        for j in range(_ROWS_W):
            pltpu.sync_copy(ones_v, hist.at[idx3.at[j, 0]], add=True)
    plsc.subcore_barrier()

    for li, hist in enumerate((hist_p, hist_n)):
        off = (cid * 2 + li) * _NB + sid * _SLICE
        pltpu.sync_copy(hist.at[pl.ds(sid * _SLICE, _SLICE)],
                        out_hbm.at[pl.ds(off, _SLICE)])


_DN_S = (((0,), (0,)), ((), ()))


def _dense_body(rows_ref, cnt_ref, s_ref, w_ref):
    i = pl.program_id(0)
    r = rows_ref[...]
    sq = jnp.sum(r * r, axis=1, keepdims=True)
    inv2 = 1.0 / jnp.maximum(jnp.sqrt(sq), _EPS)
    cnt = cnt_ref[0]
    s_part = lax.dot_general(cnt, r, _DN_S,
                             preferred_element_type=jnp.float32)
    w_part = lax.dot_general(cnt * inv2, r, _DN_S,
                             preferred_element_type=jnp.float32)

    @pl.when(i == 0)
    def _():
        s_ref[...] = s_part
        w_ref[...] = w_part

    @pl.when(i > 0)
    def _():
        s_ref[...] += s_part
        w_ref[...] += w_part


_dense = pl.pallas_call(
    _dense_body,
    grid=(_V // _BLK,),
    in_specs=[
        pl.BlockSpec((_BLK, _D), lambda i: (i, 0)),
        pl.BlockSpec((1, _BLK, 2), lambda i: (i, 0, 0)),
    ],
    out_specs=[
        pl.BlockSpec((2, _D), lambda i: (0, 0)),
        pl.BlockSpec((2, _D), lambda i: (0, 0)),
    ],
    out_shape=[
        jax.ShapeDtypeStruct((2, _D), jnp.float32),
        jax.ShapeDtypeStruct((2, _D), jnp.float32),
    ],
)


def _side_loss(s_vec, w_vec):
    c = s_vec / _N
    cnorm = jnp.maximum(jnp.sqrt(jnp.sum(c * c)), _EPS)
    mean_cos = jnp.dot(w_vec, c) / (_N * cnorm)
    return 2.0 - 2.0 * mean_cos


def kernel(embeddings, positive_nodes, negative_nodes):
    pad = jnp.full((_PAD_B - _N,), _SENT, jnp.int32)
    pos = jnp.concatenate([positive_nodes.astype(jnp.int32), pad])
    neg = jnp.concatenate([negative_nodes.astype(jnp.int32), pad])
    counts = _hist(pos.reshape(-1, 1, _IDXW), neg.reshape(-1, 1, _IDXW))
    c2 = counts.reshape(_NC, 2, _NB).sum(axis=0)[:, :_V].T
    s_out, w_out = _dense(embeddings, c2.reshape(_V // _BLK, _BLK, 2))
    pos_loss = _side_loss(s_out[0], w_out[0])
    neg_loss = _side_loss(s_out[1], w_out[1])
    return (pos_loss + neg_loss) / 2.0


# speedup vs baseline: 2.5578x; 1.0088x over previous
import functools

import jax
import jax.numpy as jnp
from jax import lax
from jax.experimental import pallas as pl
from jax.experimental.pallas import tpu as pltpu
from jax.experimental.pallas import tpu_sc as plsc

_EPS = 1e-8

_NC, _NS, _L = 2, 16, 16
_NW = _NC * _NS
_N = 50000
_V = 100000
_D = 128
_SENT = _V
_NB = 100352
_SLICE = _NB // _NS
_IDXW = 112
_ROWS_W = 14
_PAD_B = _NW * _ROWS_W * _IDXW
_BLK = 10000
_ZCH = 1568


@functools.partial(
    pl.kernel,
    mesh=plsc.VectorSubcoreMesh(core_axis_name="c", subcore_axis_name="s"),
    out_type=jax.ShapeDtypeStruct((_NC * 2 * _NB,), jnp.float32),
    scratch_types=[
        pltpu.VMEM((_ROWS_W, 1, _IDXW), jnp.int32),
        pltpu.VMEM((_IDXW,), jnp.float32),
        pltpu.VMEM((_ZCH,), jnp.float32),
        pltpu.VMEM_SHARED((_NB,), jnp.float32),
        pltpu.VMEM_SHARED((_NB,), jnp.float32),
        pltpu.SemaphoreType.DMA,
    ],
)
def _hist(pos_hbm, neg_hbm, out_hbm, idx3, ones_v, zbuf, hist_p, hist_n,
          sem):
    cid = lax.axis_index("c")
    sid = lax.axis_index("s")
    wid = sid * _NC + cid

    one = jnp.ones((_L,), jnp.float32)
    zero = jnp.zeros((_L,), jnp.float32)
    for t in range(_IDXW // _L):
        ones_v[pl.ds(t * _L, _L)] = one
    for t in range(_ZCH // _L):
        zbuf[pl.ds(t * _L, _L)] = zero

    for hist in (hist_p, hist_n):
        for q in range(_SLICE // _ZCH):
            pltpu.sync_copy(
                zbuf, hist.at[pl.ds(sid * _SLICE + q * _ZCH, _ZCH)])
    plsc.subcore_barrier()

    for hist, src_hbm in ((hist_p, pos_hbm), (hist_n, neg_hbm)):
        pltpu.sync_copy(src_hbm.at[pl.ds(wid * _ROWS_W, _ROWS_W)], idx3)
        for j in range(_ROWS_W):
            pltpu.sync_copy(ones_v, hist.at[idx3.at[j, 0]], add=True)
    plsc.subcore_barrier()

    for li, hist in enumerate((hist_p, hist_n)):
        off = (cid * 2 + li) * _NB + sid * _SLICE
        pltpu.sync_copy(hist.at[pl.ds(sid * _SLICE, _SLICE)],
                        out_hbm.at[pl.ds(off, _SLICE)])


_DN_S = (((0,), (0,)), ((), ()))


def _dense_body(rows_ref, cnt_ref, out_ref):
    i = pl.program_id(0)
    r = rows_ref[...]
    sq = jnp.dot(r * r, jnp.ones((_D, 1), jnp.float32),
                 preferred_element_type=jnp.float32)
    inv2 = jnp.where(sq >= _EPS * _EPS, lax.rsqrt(sq), 1.0 / _EPS)
    cnt = cnt_ref[0]
    lhs4 = jnp.concatenate([cnt, cnt * inv2], axis=1)
    part = lax.dot_general(lhs4, r, _DN_S,
                           preferred_element_type=jnp.float32)

    @pl.when(i == 0)
    def _():
        out_ref[...] = part

    @pl.when(i > 0)
    def _():
        out_ref[...] += part


_dense = pl.pallas_call(
    _dense_body,
    grid=(_V // _BLK,),
    in_specs=[
        pl.BlockSpec((_BLK, _D), lambda i: (i, 0)),
        pl.BlockSpec((1, _BLK, 2), lambda i: (i, 0, 0)),
    ],
    out_specs=pl.BlockSpec((4, _D), lambda i: (0, 0)),
    out_shape=jax.ShapeDtypeStruct((4, _D), jnp.float32),
)


def _side_loss(s_vec, w_vec):
    c = s_vec / _N
    cnorm = jnp.maximum(jnp.sqrt(jnp.sum(c * c)), _EPS)
    mean_cos = jnp.dot(w_vec, c) / (_N * cnorm)
    return 2.0 - 2.0 * mean_cos


def kernel(embeddings, positive_nodes, negative_nodes):
    pad = jnp.full((_PAD_B - _N,), _SENT, jnp.int32)
    pos = jnp.concatenate([positive_nodes.astype(jnp.int32), pad])
    neg = jnp.concatenate([negative_nodes.astype(jnp.int32), pad])
    counts = _hist(pos.reshape(-1, 1, _IDXW), neg.reshape(-1, 1, _IDXW))
    c2 = counts.reshape(_NC, 2, _NB).sum(axis=0)[:, :_V].T
    parts = _dense(embeddings, c2.reshape(_V // _BLK, _BLK, 2))
    pos_loss = _side_loss(parts[0], parts[2])
    neg_loss = _side_loss(parts[1], parts[3])
    return (pos_loss + neg_loss) / 2.0


# speedup vs baseline: 2.6104x; 1.0206x over previous
import functools

import jax
import jax.numpy as jnp
from jax import lax
from jax.experimental import pallas as pl
from jax.experimental.pallas import tpu as pltpu
from jax.experimental.pallas import tpu_sc as plsc

_EPS = 1e-8

_NC, _NS, _L = 2, 16, 16
_NW = _NC * _NS
_N = 50000
_PER_W = 1568
_PAD_B = _NW * _PER_W
_CHUNK = 112
_NCHUNK = _PER_W // _CHUNK
_TAIL_VALID = _N - (_NW - 1) * _PER_W
_NGRP = _CHUNK // _L
_D = 128
_KREG = _D // _L


_GATHER_DNUMS = lax.GatherDimensionNumbers(
    offset_dims=(), collapsed_slice_dims=(0,), start_index_map=(0,))


def _lane_gather(x, idx):
    return lax.gather(x, idx.reshape(_L, 1), _GATHER_DNUMS, (1,),
                      mode=lax.GatherScatterMode.PROMISE_IN_BOUNDS)


def _hsum_splat(x):
    lanes = lax.iota(jnp.int32, _L)
    for d in (8, 4, 2, 1):
        x = x + _lane_gather(x, lanes ^ d)
    return x


def _bcast_lane(x, i):
    return _lane_gather(x, jnp.full((_L,), i, dtype=jnp.int32))


def _nr_rsqrt(s):
    s1 = s * 2.0**24
    y_scale = jnp.full((_L,), 2.0**12, dtype=jnp.float32)
    for e in (32, 16, 8, 4, 2, 1):
        big = s1 >= 2.0**e
        s1 = jnp.where(big, s1 * 2.0**-e, s1)
        y_scale = y_scale * jnp.where(big, jnp.float32(2.0 ** (-e / 2)),
                                      jnp.float32(1.0))
    y = 2.0 / (1.0 + s1)
    for _ in range(3):
        y = y * (1.5 - 0.5 * s1 * y * y)
    return y * y_scale


@functools.partial(
    pl.kernel,
    mesh=plsc.VectorSubcoreMesh(core_axis_name="c", subcore_axis_name="s"),
    out_type=jax.ShapeDtypeStruct((_NW, 4, _D), jnp.float32),
    scratch_types=[
        pltpu.VMEM((_PER_W,), jnp.int32),
        pltpu.VMEM((_PER_W,), jnp.int32),
        pltpu.VMEM((_CHUNK, _D), jnp.float32),
        pltpu.VMEM((_CHUNK, _D), jnp.float32),
        pltpu.VMEM((4, _D), jnp.float32),
        pltpu.VMEM((_D,), jnp.float32),
        pltpu.SemaphoreType.DMA,
        pltpu.SemaphoreType.DMA,
    ],
)
def _partials(emb_hbm, pos_hbm, neg_hbm, out_hbm, idx_p, idx_n, buf0, buf1,
              out_v, wacc, sem0, sem1):
    wid = lax.axis_index("s") * _NC + lax.axis_index("c")
    base = wid * _PER_W

    full = base + _PER_W <= _N
    for src_hbm, idx_v in ((pos_hbm, idx_p), (neg_hbm, idx_n)):

        @pl.when(full)
        def _(src_hbm=src_hbm, idx_v=idx_v):
            pltpu.sync_copy(src_hbm.at[pl.ds(base, _PER_W)], idx_v)

        @pl.when(jnp.logical_not(full))
        def _(src_hbm=src_hbm, idx_v=idx_v):
            pltpu.sync_copy(src_hbm.at[pl.ds(base, _TAIL_VALID)],
                            idx_v.at[pl.ds(0, _TAIL_VALID)])
            zi = jnp.zeros((_L,), jnp.int32)
            for t in range((_PER_W - _TAIL_VALID) // _L):
                idx_v[pl.ds(_TAIL_VALID + t * _L, _L)] = zi

    def src(idx_v, cc):
        return emb_hbm.at[idx_v.at[pl.ds(cc * _CHUNK, _CHUNK)]]

    def fire(idx_v, cc, buf, sem):
        @pl.when(cc < _NCHUNK)
        def _():
            pltpu.async_copy(src(idx_v, jnp.minimum(cc, _NCHUNK - 1)),
                             buf, sem)

    def drain(idx_v, cc, buf, sem):
        pltpu.make_async_copy(src(idx_v, cc), buf, sem).wait()

    def process(buf, accs, cc):
        first_pad = _N - base - cc * _CHUNK

        @pl.when(first_pad < _CHUNK)
        def _():
            z = jnp.zeros((_L,), jnp.float32)

            def zbody(i, carry):
                for k in range(_KREG):
                    buf[i, pl.ds(k * _L, _L)] = z
                return carry

            lax.fori_loop(jnp.maximum(first_pad, 0), _CHUNK, zbody, 0)

        lanes = lax.iota(jnp.int32, _L)

        def gbody(g, accs):
            base_r = g * _L

            def p1(i, sqpack):
                v = [buf[base_r + i, pl.ds(k * _L, _L)]
                     for k in range(_KREG)]
                sq = v[0] * v[0]
                for k in range(1, _KREG):
                    sq = sq + v[k] * v[k]
                tot = _hsum_splat(sq)
                return jnp.where(lanes == i, tot, sqpack)

            sqpack = lax.fori_loop(0, _L, p1, jnp.zeros((_L,), jnp.float32),
                                   unroll=4)
            norm = sqpack * _nr_rsqrt(sqpack)
            w_grp = 1.0 / jnp.maximum(norm, _EPS)

            def p2(i, accs):
                wv = _bcast_lane(w_grp, i)
                v = [buf[base_r + i, pl.ds(k * _L, _L)]
                     for k in range(_KREG)]
                for k in range(_KREG):
                    plsc.addupdate(wacc.at[pl.ds(k * _L, _L)], wv * v[k])
                return tuple(accs[k] + v[k] for k in range(_KREG))

            return lax.fori_loop(0, _L, p2, accs, unroll=4)

        return lax.fori_loop(0, _NGRP, gbody, accs)

    for li, idx_v in enumerate((idx_p, idx_n)):
        fire(idx_v, 0, buf0, sem0)
        fire(idx_v, 1, buf1, sem1)

        def pipe_body(g, accs, idx_v=idx_v):
            c0 = 2 * g
            c1 = 2 * g + 1
            drain(idx_v, c0, buf0, sem0)
            accs = process(buf0, accs, c0)
            fire(idx_v, c0 + 2, buf0, sem0)
            drain(idx_v, c1, buf1, sem1)
            accs = process(buf1, accs, c1)
            fire(idx_v, c1 + 2, buf1, sem1)
            return accs

        zf = jnp.zeros((_L,), jnp.float32)
        for k in range(_KREG):
            wacc[pl.ds(k * _L, _L)] = zf
        accs = tuple(zf for _ in range(_KREG))
        accs = lax.fori_loop(0, _NCHUNK // 2, pipe_body, accs)
        for k in range(_KREG):
            out_v[2 * li + 0, pl.ds(k * _L, _L)] = accs[k]
            out_v[2 * li + 1, pl.ds(k * _L, _L)] = wacc[pl.ds(k * _L, _L)]

    pltpu.sync_copy(out_v, out_hbm.at[wid])


def _side_loss(s_vec, w_vec):
    c = s_vec / _N
    cnorm = jnp.maximum(jnp.sqrt(jnp.sum(c * c)), _EPS)
    mean_cos = jnp.dot(w_vec, c) / (_N * cnorm)
    return 2.0 - 2.0 * mean_cos


def kernel(embeddings, positive_nodes, negative_nodes):
    parts = _partials(embeddings, positive_nodes.astype(jnp.int32),
                      negative_nodes.astype(jnp.int32))
    tot = jnp.sum(parts, axis=0)
    pos_loss = _side_loss(tot[0], tot[1])
    neg_loss = _side_loss(tot[2], tot[3])
    return (pos_loss + neg_loss) / 2.0


# speedup vs baseline: 2.9345x; 1.1241x over previous
import functools

import jax
import jax.numpy as jnp
from jax import lax
from jax.experimental import pallas as pl
from jax.experimental.pallas import tpu as pltpu
from jax.experimental.pallas import tpu_sc as plsc

_EPS = 1e-8

_NC, _NS, _L = 2, 16, 16
_NW = _NC * _NS
_N = 50000
_PER_W = 1568
_PAD_B = _NW * _PER_W
_CHUNK = 112
_NCHUNK = _PER_W // _CHUNK
_TAIL_VALID = _N - (_NW - 1) * _PER_W
_NGRP = _CHUNK // _L
_D = 128
_KREG = _D // _L


_GATHER_DNUMS = lax.GatherDimensionNumbers(
    offset_dims=(), collapsed_slice_dims=(0,), start_index_map=(0,))


def _lane_gather(x, idx):
    return lax.gather(x, idx.reshape(_L, 1), _GATHER_DNUMS, (1,),
                      mode=lax.GatherScatterMode.PROMISE_IN_BOUNDS)


def _hsum_splat(x):
    lanes = lax.iota(jnp.int32, _L)
    for d in (8, 4, 2, 1):
        x = x + _lane_gather(x, lanes ^ d)
    return x


def _bcast_lane(x, i):
    return _lane_gather(x, jnp.full((_L,), i, dtype=jnp.int32))


def _nr_rsqrt(s):
    s1 = s * 2.0**24
    y_scale = jnp.full((_L,), 2.0**12, dtype=jnp.float32)
    for e in (32, 16, 8, 4, 2, 1):
        big = s1 >= 2.0**e
        s1 = jnp.where(big, s1 * 2.0**-e, s1)
        y_scale = y_scale * jnp.where(big, jnp.float32(2.0 ** (-e / 2)),
                                      jnp.float32(1.0))
    y = 2.0 / (1.0 + s1)
    for _ in range(3):
        y = y * (1.5 - 0.5 * s1 * y * y)
    return y * y_scale


@functools.partial(
    pl.kernel,
    mesh=plsc.VectorSubcoreMesh(core_axis_name="c", subcore_axis_name="s"),
    out_type=jax.ShapeDtypeStruct((_NW, 4, _D), jnp.float32),
    scratch_types=[
        pltpu.VMEM((_PER_W,), jnp.int32),
        pltpu.VMEM((_PER_W,), jnp.int32),
        pltpu.VMEM((_CHUNK, _D), jnp.float32),
        pltpu.VMEM((_CHUNK, _D), jnp.float32),
        pltpu.VMEM((4, _D), jnp.float32),
        pltpu.SemaphoreType.DMA,
        pltpu.SemaphoreType.DMA,
    ],
)
def _partials(emb_hbm, pos_hbm, neg_hbm, out_hbm, idx_p, idx_n, buf0, buf1,
              out_v, sem0, sem1):
    wid = lax.axis_index("s") * _NC + lax.axis_index("c")
    base = wid * _PER_W

    full = base + _PER_W <= _N
    for src_hbm, idx_v in ((pos_hbm, idx_p), (neg_hbm, idx_n)):

        @pl.when(full)
        def _(src_hbm=src_hbm, idx_v=idx_v):
            pltpu.sync_copy(src_hbm.at[pl.ds(base, _PER_W)], idx_v)

        @pl.when(jnp.logical_not(full))
        def _(src_hbm=src_hbm, idx_v=idx_v):
            pltpu.sync_copy(src_hbm.at[pl.ds(base, _TAIL_VALID)],
                            idx_v.at[pl.ds(0, _TAIL_VALID)])
            zi = jnp.zeros((_L,), jnp.int32)
            for t in range((_PER_W - _TAIL_VALID) // _L):
                idx_v[pl.ds(_TAIL_VALID + t * _L, _L)] = zi

    def src(idx_v, cc):
        return emb_hbm.at[idx_v.at[pl.ds(cc * _CHUNK, _CHUNK)]]

    def fire(idx_v, cc, buf, sem):
        @pl.when(cc < _NCHUNK)
        def _():
            pltpu.async_copy(src(idx_v, jnp.minimum(cc, _NCHUNK - 1)),
                             buf, sem)

    def drain(idx_v, cc, buf, sem):
        pltpu.make_async_copy(src(idx_v, cc), buf, sem).wait()

    def process(buf, accs, cc):
        first_pad = _N - base - cc * _CHUNK

        @pl.when(first_pad < _CHUNK)
        def _():
            z = jnp.zeros((_L,), jnp.float32)

            def zbody(i, carry):
                for k in range(_KREG):
                    buf[i, pl.ds(k * _L, _L)] = z
                return carry

            lax.fori_loop(jnp.maximum(first_pad, 0), _CHUNK, zbody, 0)

        lanes = lax.iota(jnp.int32, _L)

        def gbody(g, accs):
            base_r = g * _L

            def p1(i, sqpack):
                v = [buf[base_r + i, pl.ds(k * _L, _L)]
                     for k in range(_KREG)]
                sq = v[0] * v[0]
                for k in range(1, _KREG):
                    sq = sq + v[k] * v[k]
                tot = _hsum_splat(sq)
                return jnp.where(lanes == i, tot, sqpack)

            sqpack = lax.fori_loop(0, _L, p1, jnp.zeros((_L,), jnp.float32),
                                   unroll=4)
            norm = sqpack * _nr_rsqrt(sqpack)
            w_grp = 1.0 / jnp.maximum(norm, _EPS)

            def p2(i, accs):
                wv = _bcast_lane(w_grp, i)
                v = [buf[base_r + i, pl.ds(k * _L, _L)]
                     for k in range(_KREG)]
                a_s = tuple(accs[k] + v[k] for k in range(_KREG))
                a_w = tuple(accs[_KREG + k] + wv * v[k]
                            for k in range(_KREG))
                return a_s + a_w

            return lax.fori_loop(0, _L, p2, accs, unroll=4)

        return lax.fori_loop(0, _NGRP, gbody, accs)

    for li, idx_v in enumerate((idx_p, idx_n)):
        fire(idx_v, 0, buf0, sem0)
        fire(idx_v, 1, buf1, sem1)

        def pipe_body(g, accs, idx_v=idx_v):
            c0 = 2 * g
            c1 = 2 * g + 1
            drain(idx_v, c0, buf0, sem0)
            accs = process(buf0, accs, c0)
            fire(idx_v, c0 + 2, buf0, sem0)
            drain(idx_v, c1, buf1, sem1)
            accs = process(buf1, accs, c1)
            fire(idx_v, c1 + 2, buf1, sem1)
            return accs

        accs = tuple(jnp.zeros((_L,), jnp.float32) for _ in range(2 * _KREG))
        accs = lax.fori_loop(0, _NCHUNK // 2, pipe_body, accs)
        for k in range(_KREG):
            out_v[2 * li + 0, pl.ds(k * _L, _L)] = accs[k]
            out_v[2 * li + 1, pl.ds(k * _L, _L)] = accs[_KREG + k]

    pltpu.sync_copy(out_v, out_hbm.at[wid])


def _side_loss(s_vec, w_vec):
    c = s_vec / _N
    cnorm = jnp.maximum(jnp.sqrt(jnp.sum(c * c)), _EPS)
    mean_cos = jnp.dot(w_vec, c) / (_N * cnorm)
    return 2.0 - 2.0 * mean_cos


def kernel(embeddings, positive_nodes, negative_nodes):
    parts = _partials(embeddings, positive_nodes.astype(jnp.int32),
                      negative_nodes.astype(jnp.int32))
    tot = jnp.sum(parts, axis=0)
    pos_loss = _side_loss(tot[0], tot[1])
    neg_loss = _side_loss(tot[2], tot[3])
    return (pos_loss + neg_loss) / 2.0
